# baseline (speedup 1.0000x reference)
import functools

import jax
import jax.numpy as jnp
import numpy as np
from jax import lax
from jax.experimental import pallas as pl
from jax.experimental.pallas import tpu as pltpu

_N = 20000
_NP = 20480
_C = 8
_MAX_DET = 100
_NMS_THR = 0.5
_THR_LOGIT = float(np.log(0.01) - np.log(0.99))
_BIG = np.int32(2**30)
_NEG = -jnp.inf


_CH = 1024
_NCH = _NP // _CH


def _nms_kernel(bbT_ref, lg_ref, bbr_ref, rot_ref, tr_ref,
                boxes_o, scores_o, labels_o, rot_o, tr_o,
                s_ref, x1_ref, y1_ref, x2_ref, y2_ref, ar_ref, io_ref):
    bx1 = jnp.broadcast_to(bbT_ref[0:1, :], (_C, _NP))
    by1 = jnp.broadcast_to(bbT_ref[1:2, :], (_C, _NP))
    bx2 = jnp.broadcast_to(bbT_ref[2:3, :], (_C, _NP))
    by2 = jnp.broadcast_to(bbT_ref[3:4, :], (_C, _NP))
    x1_ref[...] = bx1
    y1_ref[...] = by1
    x2_ref[...] = bx2
    y2_ref[...] = by2
    ar_ref[...] = (bx2 - bx1) * (by2 - by1)

    io_ref[...] = lax.broadcasted_iota(jnp.int32, (_C, _NP), 1)
    lane = lax.broadcasted_iota(jnp.int32, (_C, 128), 1)

    lg = lg_ref[...]
    s0 = jnp.where(lg > _THR_LOGIT, lg, _NEG)
    s_ref[...] = s0
    m0 = jnp.max(s0, axis=1, keepdims=True)

    def nms_step(i, carry):
        cs, ci, m = carry
        iacc = jnp.full((_C, 128), _BIG, jnp.int32)
        for k in range(_NCH):
            sl = pl.ds(k * _CH, _CH)
            cnd = jnp.where(s_ref[:, sl] == m, io_ref[:, sl], _BIG)
            iacc = jnp.minimum(
                iacc, jnp.min(cnd.reshape(_C, _CH // 128, 128), axis=1))
        idx = jnp.min(iacc, axis=1)
        at_i = lane == i
        cs = jnp.where(at_i, m, cs)
        ci = jnp.where(at_i, idx.reshape(_C, 1), ci)
        rows = [bbr_ref[pl.ds(idx[c], 1), :] for c in range(_C)]
        sel = jnp.concatenate(rows, axis=0)
        sx1 = sel[:, 0:1]
        sy1 = sel[:, 1:2]
        sx2 = sel[:, 2:3]
        sy2 = sel[:, 3:4]
        sb = (sx2 - sx1) * (sy2 - sy1) + 1e-8
        macc = jnp.full((_C, 128), _NEG, jnp.float32)
        for k in range(_NCH):
            sl = pl.ds(k * _CH, _CH)
            s = s_ref[:, sl]
            iw = jnp.minimum(sx2, x2_ref[:, sl]) - jnp.maximum(sx1, x1_ref[:, sl])
            ih = jnp.minimum(sy2, y2_ref[:, sl]) - jnp.maximum(sy1, y1_ref[:, sl])
            inter = iw * jnp.maximum(ih, 0.0)
            kill = inter + inter + inter > ar_ref[:, sl] + sb
            s_new = jnp.where(kill, _NEG, s)
            s_ref[:, sl] = s_new
            macc = jnp.maximum(
                macc, jnp.max(s_new.reshape(_C, _CH // 128, 128), axis=1))
        return (cs, ci, jnp.max(macc, axis=1, keepdims=True))

    cs0 = jnp.full((_C, 128), _NEG, jnp.float32)
    ci0 = jnp.zeros((_C, 128), jnp.int32)
    cs_f, ci_f, _ = lax.fori_loop(0, _MAX_DET, nms_step, (cs0, ci0, m0),
                                  unroll=False)
    ci = ci_f

    flat = lax.broadcasted_iota(jnp.int32, (_C, 128), 0) * 128 + lane

    def merge_step(p, cs):
        gmax = jnp.max(cs)
        fpos = jnp.where(cs == gmax, flat, _BIG)
        fp = jnp.min(fpos)
        hit = fpos == fp
        cls = fp >> 7
        bidx = jnp.clip(jnp.min(jnp.where(hit, ci, _BIG)), 0, _N - 1)
        valid = gmax > -1e30
        cs = jnp.where(hit, _NEG, cs)
        scores_o[pl.ds(p, 1), :] = gmax.reshape(1, 1)
        labels_o[pl.ds(p, 1), :] = jnp.where(valid, cls, -1).reshape(1, 1)
        brow = bbr_ref[pl.ds(bidx, 1), :]
        boxes_o[pl.ds(p, 1), :] = jnp.where(valid, brow, -1.0)
        rrow = rot_ref[pl.ds(bidx, 1), :]
        rot_o[pl.ds(p, 1), :] = jnp.where(valid, rrow, -1.0)
        trow = tr_ref[pl.ds(bidx, 1), :]
        tr_o[pl.ds(p, 1), :] = jnp.where(valid, trow, -1.0)
        return cs

    lax.fori_loop(0, _MAX_DET, merge_step, cs_f, unroll=False)

    sl = scores_o[...]
    scores_o[...] = jnp.where(sl > -1e30, jax.nn.sigmoid(sl), -1.0)


@jax.jit
def kernel(bboxes, classification, translation, rotation):
    bb = bboxes[0]
    pad = _NP - _N
    bbT = jnp.pad(bb.T, ((0, 0), (0, pad)))
    lg = jnp.pad(classification[0].T, ((0, 0), (0, pad)),
                 constant_values=-1e9)
    rot = rotation[0]
    tr = translation[0]
    boxes_o, scores_o, labels_o, rot_o, tr_o = pl.pallas_call(
        _nms_kernel,
        out_shape=(
            jax.ShapeDtypeStruct((128, 4), jnp.float32),
            jax.ShapeDtypeStruct((128, 1), jnp.float32),
            jax.ShapeDtypeStruct((128, 1), jnp.int32),
            jax.ShapeDtypeStruct((128, 3), jnp.float32),
            jax.ShapeDtypeStruct((128, 3), jnp.float32),
        ),
        scratch_shapes=[pltpu.VMEM((_C, _NP), jnp.float32)] * 6
        + [pltpu.VMEM((_C, _NP), jnp.int32)],
    )(bbT, lg, bb, rot, tr)
    return (boxes_o[:_MAX_DET][None],
            scores_o[:_MAX_DET, 0][None],
            labels_o[:_MAX_DET, 0][None],
            rot_o[:_MAX_DET][None],
            tr_o[:_MAX_DET][None])


# speedup vs baseline: 1.2977x; 1.2977x over previous
import functools

import jax
import jax.numpy as jnp
import numpy as np
from jax import lax
from jax.experimental import pallas as pl
from jax.experimental.pallas import tpu as pltpu

_N = 20000
_NP = 20480
_C = 8
_MAX_DET = 100
_NMS_THR = 0.5
_THR_LOGIT = float(np.log(0.01) - np.log(0.99))
_BIG = np.int32(2**30)
_NEG = -jnp.inf


_CH = 1024
_NCH = _NP // _CH


def _fold_lanes(x, op):
    w = x.shape[1]
    while w > 128:
        w //= 2
        x = op(x[:, :w], x[:, w:])
    return x


def _nms_kernel(bbT_ref, lg_ref, bbr_ref, rot_ref, tr_ref,
                boxes_o, scores_o, labels_o, rot_o, tr_o,
                s_ref, x1_ref, y1_ref, x2_ref, y2_ref, ar_ref, io_ref):
    bx1 = jnp.broadcast_to(bbT_ref[0:1, :], (_C, _NP))
    by1 = jnp.broadcast_to(bbT_ref[1:2, :], (_C, _NP))
    bx2 = jnp.broadcast_to(bbT_ref[2:3, :], (_C, _NP))
    by2 = jnp.broadcast_to(bbT_ref[3:4, :], (_C, _NP))
    x1_ref[...] = bx1
    y1_ref[...] = by1
    x2_ref[...] = bx2
    y2_ref[...] = by2
    ar_ref[...] = (bx2 - bx1) * (by2 - by1)

    io_ref[...] = lax.broadcasted_iota(jnp.int32, (_C, _NP), 1)
    lane = lax.broadcasted_iota(jnp.int32, (_C, 128), 1)

    lg = lg_ref[...]
    s0 = jnp.where(lg > _THR_LOGIT, lg, _NEG)
    s_ref[...] = s0
    m0 = jnp.max(s0, axis=1, keepdims=True)

    def nms_step(i, carry):
        cs, ci, m = carry
        iacc = jnp.full((_C, 128), _BIG, jnp.int32)
        for k in range(_NCH):
            sl = pl.ds(k * _CH, _CH)
            cnd = jnp.where(s_ref[:, sl] == m, io_ref[:, sl], _BIG)
            iacc = jnp.minimum(iacc, _fold_lanes(cnd, jnp.minimum))
        idx = jnp.min(iacc, axis=1)
        at_i = lane == i
        cs = jnp.where(at_i, m, cs)
        ci = jnp.where(at_i, idx.reshape(_C, 1), ci)
        rows = [bbr_ref[pl.ds(idx[c], 1), :] for c in range(_C)]
        sel = jnp.concatenate(rows, axis=0)
        sx1 = sel[:, 0:1]
        sy1 = sel[:, 1:2]
        sx2 = sel[:, 2:3]
        sy2 = sel[:, 3:4]
        sb = (sx2 - sx1) * (sy2 - sy1) + 1e-8
        macc = jnp.full((_C, 128), _NEG, jnp.float32)
        for k in range(_NCH):
            sl = pl.ds(k * _CH, _CH)
            s = s_ref[:, sl]
            iw = jnp.minimum(sx2, x2_ref[:, sl]) - jnp.maximum(sx1, x1_ref[:, sl])
            ih = jnp.minimum(sy2, y2_ref[:, sl]) - jnp.maximum(sy1, y1_ref[:, sl])
            inter = iw * jnp.maximum(ih, 0.0)
            kill = inter + inter + inter > ar_ref[:, sl] + sb
            s_new = jnp.where(kill, _NEG, s)
            s_ref[:, sl] = s_new
            macc = jnp.maximum(macc, _fold_lanes(s_new, jnp.maximum))
        return (cs, ci, jnp.max(macc, axis=1, keepdims=True))

    cs0 = jnp.full((_C, 128), _NEG, jnp.float32)
    ci0 = jnp.zeros((_C, 128), jnp.int32)
    cs_f, ci_f, _ = lax.fori_loop(0, _MAX_DET, nms_step, (cs0, ci0, m0),
                                  unroll=False)
    ci = ci_f

    flat = lax.broadcasted_iota(jnp.int32, (_C, 128), 0) * 128 + lane

    def merge_step(p, cs):
        gmax = jnp.max(cs)
        fpos = jnp.where(cs == gmax, flat, _BIG)
        fp = jnp.min(fpos)
        hit = fpos == fp
        cls = fp >> 7
        bidx = jnp.clip(jnp.min(jnp.where(hit, ci, _BIG)), 0, _N - 1)
        valid = gmax > -1e30
        cs = jnp.where(hit, _NEG, cs)
        scores_o[pl.ds(p, 1), :] = gmax.reshape(1, 1)
        labels_o[pl.ds(p, 1), :] = jnp.where(valid, cls, -1).reshape(1, 1)
        brow = bbr_ref[pl.ds(bidx, 1), :]
        boxes_o[pl.ds(p, 1), :] = jnp.where(valid, brow, -1.0)
        rrow = rot_ref[pl.ds(bidx, 1), :]
        rot_o[pl.ds(p, 1), :] = jnp.where(valid, rrow, -1.0)
        trow = tr_ref[pl.ds(bidx, 1), :]
        tr_o[pl.ds(p, 1), :] = jnp.where(valid, trow, -1.0)
        return cs

    lax.fori_loop(0, _MAX_DET, merge_step, cs_f, unroll=False)

    sl = scores_o[...]
    scores_o[...] = jnp.where(sl > -1e30, jax.nn.sigmoid(sl), -1.0)


@jax.jit
def kernel(bboxes, classification, translation, rotation):
    bb = bboxes[0]
    pad = _NP - _N
    bbT = jnp.pad(bb.T, ((0, 0), (0, pad)))
    lg = jnp.pad(classification[0].T, ((0, 0), (0, pad)),
                 constant_values=-1e9)
    rot = rotation[0]
    tr = translation[0]
    boxes_o, scores_o, labels_o, rot_o, tr_o = pl.pallas_call(
        _nms_kernel,
        out_shape=(
            jax.ShapeDtypeStruct((128, 4), jnp.float32),
            jax.ShapeDtypeStruct((128, 1), jnp.float32),
            jax.ShapeDtypeStruct((128, 1), jnp.int32),
            jax.ShapeDtypeStruct((128, 3), jnp.float32),
            jax.ShapeDtypeStruct((128, 3), jnp.float32),
        ),
        scratch_shapes=[pltpu.VMEM((_C, _NP), jnp.float32)] * 6
        + [pltpu.VMEM((_C, _NP), jnp.int32)],
    )(bbT, lg, bb, rot, tr)
    return (boxes_o[:_MAX_DET][None],
            scores_o[:_MAX_DET, 0][None],
            labels_o[:_MAX_DET, 0][None],
            rot_o[:_MAX_DET][None],
            tr_o[:_MAX_DET][None])
